# baseline (speedup 1.0000x reference)
import functools

import jax
import jax.numpy as jnp
from jax import lax
from jax.experimental import pallas as pl
from jax.experimental.pallas import tpu as pltpu
from jax.experimental.pallas import tpu_sc as plsc

RADIUS = 0.2
NSAMPLE = 32
EPS = 1e-5
NUM_POINTS = 1024

_B, _N, _C = 4, 8192, 64
_S = NUM_POINTS
_D = 128
_NSC = 32
_GRP_PER_B = _NSC // _B
_S_PER_W = _S // _GRP_PER_B
_NCHUNK = _N // 16
_ROWS_PER_W = _S_PER_W * NSAMPLE
_GCHUNK = 128
_NGCH = _ROWS_PER_W // _GCHUNK
_P = _B * _S * NSAMPLE

_NCORES = 2
_NSUBCORES = 16
_UNROLL = 32
_MGRP = 16


def _bf16_round(x):
    u = lax.bitcast_convert_type(x, jnp.int32)
    lsb = jnp.bitwise_and(lax.shift_right_logical(u, 16), jnp.int32(1))
    r = u + lsb + jnp.int32(0x7FFF)
    return lax.bitcast_convert_type(jnp.bitwise_and(r, jnp.int32(-65536)),
                                    jnp.float32)


def _sc_body(xyzT_hbm, T_hbm, grouped_hbm,
             xv, yv, zv, xbv, ybv, zbv, pnv, idxv, rows0, rows1, sem0, sem1):
    wid = lax.axis_index("s") * _NCORES + lax.axis_index("c")
    b = wid // _GRP_PER_B
    g = wid % _GRP_PER_B
    pltpu.sync_copy(xyzT_hbm.at[pl.ds((b * 3 + 0) * _N, _N)], xv)
    pltpu.sync_copy(xyzT_hbm.at[pl.ds((b * 3 + 1) * _N, _N)], yv)
    pltpu.sync_copy(xyzT_hbm.at[pl.ds((b * 3 + 2) * _N, _N)], zv)
    s0 = g * _S_PER_W
    bN = b * _N
    lane = lax.iota(jnp.int32, 16)
    zero16 = jnp.zeros((16,), jnp.int32)
    r2 = jnp.full((16,), jnp.float32(RADIUS * RADIUS), jnp.float32)

    def prep(i, carry):
        x = xv[pl.ds(i * 16, 16)]
        y = yv[pl.ds(i * 16, 16)]
        z = zv[pl.ds(i * 16, 16)]
        pnv[pl.ds(i * 16, 16)] = x * x + y * y + z * z
        xbv[pl.ds(i * 16, 16)] = _bf16_round(x)
        ybv[pl.ds(i * 16, 16)] = _bf16_round(y)
        zbv[pl.ds(i * 16, 16)] = _bf16_round(z)
        return carry

    lax.fori_loop(0, _NCHUNK, prep, jnp.int32(0))

    def per_centroid(ci, carry):
        s = s0 + ci
        base = ci * NSAMPLE
        idxv[pl.ds(base, 16)] = zero16
        idxv[pl.ds(base + 16, 16)] = zero16
        calign = s0 + (ci // 16) * 16
        lsel = lane == (s - calign)
        zf = jnp.zeros((16,), jnp.float32)

        def extract(ref):
            v = ref[pl.ds(calign, 16)]
            return jnp.sum(jnp.where(lsel, v, zf))

        cx, cy, cz = extract(xv), extract(yv), extract(zv)
        cn = jnp.full((16,), cx * cx + cy * cy + cz * cz, jnp.float32)
        cbx = jnp.full((16,), 2.0 * extract(xbv), jnp.float32)
        cby = jnp.full((16,), 2.0 * extract(ybv), jnp.float32)
        cbz = jnp.full((16,), 2.0 * extract(zbv), jnp.float32)

        def cond(c):
            i, cnt = c
            return jnp.logical_and(i < _NCHUNK, cnt < NSAMPLE)

        def step(c):
            i, cnt = c
            for h in range(_UNROLL // _MGRP):
                ms = []
                for u in range(_MGRP):
                    ii = i + h * _MGRP + u
                    px = xbv[pl.ds(ii * 16, 16)]
                    py = ybv[pl.ds(ii * 16, 16)]
                    pz = zbv[pl.ds(ii * 16, 16)]
                    pn = pnv[pl.ds(ii * 16, 16)]
                    dot2 = (cbx * px + cby * py) + cbz * pz
                    d2 = (cn + pn) - dot2
                    ms.append(d2 < r2)
                for u in range(_MGRP):
                    ii = i + h * _MGRP + u
                    n = jnp.sum(ms[u].astype(jnp.int32))
                    plsc.store_compressed(idxv.at[pl.ds(base + cnt, 16)],
                                          lane + ii * 16, mask=ms[u])
                    cnt = cnt + n
            return (i + _UNROLL, cnt)

        _, cnt = lax.while_loop(cond, step, (jnp.int32(0), jnp.int32(0)))
        cntv = jnp.full((16,), cnt, jnp.int32)
        v0 = idxv[pl.ds(base, 16)]
        v1 = idxv[pl.ds(base + 16, 16)]
        zi = jnp.zeros((16,), jnp.int32)
        first = jnp.full((16,), jnp.sum(jnp.where(lane == 0, v0, zi)),
                         jnp.int32)
        bNv = jnp.full((16,), bN, jnp.int32)
        idxv[pl.ds(base, 16)] = jnp.where(lane < cntv, v0, first) + bNv
        idxv[pl.ds(base + 16, 16)] = jnp.where(lane + 16 < cntv, v1, first) \
            + bNv
        return carry

    lax.fori_loop(0, _S_PER_W, per_centroid, jnp.int32(0))

    row_base = wid * _ROWS_PER_W
    bufs = (rows0, rows1)
    sems = (sem0, sem1)

    def fire(j):
        return pltpu.async_copy(
            T_hbm.at[idxv.at[pl.ds(j * _GCHUNK, _GCHUNK)]],
            bufs[j % 2], sems[j % 2])

    dmas = [None, None]
    dmas[0] = fire(0)
    for j in range(_NGCH):
        dmas[j % 2].wait()
        if j + 1 < _NGCH:
            dmas[(j + 1) % 2] = fire(j + 1)
        pltpu.sync_copy(bufs[j % 2],
                        grouped_hbm.at[pl.ds(row_base + j * _GCHUNK,
                                             _GCHUNK)])


@jax.jit
def _sc_group(xyzT, T):
    mesh = plsc.VectorSubcoreMesh(core_axis_name="c", subcore_axis_name="s",
                                  num_cores=_NCORES, num_subcores=_NSUBCORES)
    f = functools.partial(
        pl.kernel,
        mesh=mesh,
        compiler_params=pltpu.CompilerParams(needs_layout_passes=False),
        out_type=jax.ShapeDtypeStruct((_B * _S * NSAMPLE, _D), jnp.float32),
        scratch_types=[
            pltpu.VMEM((_N,), jnp.float32),
            pltpu.VMEM((_N,), jnp.float32),
            pltpu.VMEM((_N,), jnp.float32),
            pltpu.VMEM((_N,), jnp.float32),
            pltpu.VMEM((_N,), jnp.float32),
            pltpu.VMEM((_N,), jnp.float32),
            pltpu.VMEM((_N,), jnp.float32),
            pltpu.VMEM((_ROWS_PER_W + 16 * _UNROLL + 32,), jnp.int32),
            pltpu.VMEM((_GCHUNK, _D), jnp.float32),
            pltpu.VMEM((_GCHUNK, _D), jnp.float32),
            pltpu.SemaphoreType.DMA,
            pltpu.SemaphoreType.DMA,
        ],
    )(_sc_body)
    return f(xyzT, T)


_RBLK = 4096
_CBLK = _RBLK // NSAMPLE
_NBLK = (_B * _S * NSAMPLE) // _RBLK


def _stats(W, bias, gamma, beta, G2, ssum):
    hp = jax.lax.Precision.HIGHEST
    sW = jnp.dot(ssum, W, precision=hp,
                 preferred_element_type=jnp.float32)
    GW = jnp.dot(G2, W, precision=hp,
                 preferred_element_type=jnp.float32)
    diag = jnp.sum(W * GW, axis=0, keepdims=True)
    Pf = jnp.float32(_P)
    mean = sW / Pf + bias
    ey2 = (diag + 2.0 * bias * sW) / Pf + bias * bias
    var = ey2 - mean * mean
    scale = gamma * jax.lax.rsqrt(var + jnp.float32(EPS))
    shift = beta - mean * scale
    return scale, shift


def _mlp_body(rows, E, W1, b1, g1, be1, W2, b2, g2, be2, out, Gs, Hs, s2s):
    p = pl.program_id(0)
    x = (rows[...].reshape(_CBLK, NSAMPLE, _D)
         - E[...][:, None, :]).reshape(_RBLK, _D)

    @pl.when(jnp.logical_and(p == 0, pl.program_id(1) == 0))
    def _():
        Gs[...] = jnp.zeros_like(Gs)

    @pl.when(p == 0)
    def _():
        Gs[...] += lax.dot_general(x, x, (((0,), (0,)), ((), ())),
                                   preferred_element_type=jnp.float32)

    @pl.when(jnp.logical_and(p == 1, pl.program_id(1) == 0))
    def _():
        Hs[...] = jnp.zeros_like(Hs)
        s2s[...] = jnp.zeros_like(s2s)

    def _a1(G):
        sc1, sh1 = _stats(W1[...], b1[...], g1[...], be1[...],
                          G, G[_D - 1:_D, :])
        y1 = jnp.dot(x, W1[...], preferred_element_type=jnp.float32) + b1[...]
        return jnp.maximum(y1 * sc1 + sh1, 0.0)

    @pl.when(p == 1)
    def _():
        a1 = _a1(Gs[...])
        Hs[...] += lax.dot_general(a1, a1, (((0,), (0,)), ((), ())),
                                   preferred_element_type=jnp.float32)
        s2s[...] += jnp.sum(a1, axis=0, keepdims=True)

    @pl.when(p == 2)
    def _():
        a1 = _a1(Gs[...])
        sc2, sh2 = _stats(W2[...], b2[...], g2[...], be2[...],
                          Hs[...], s2s[...])
        y2 = jnp.dot(a1, W2[...], preferred_element_type=jnp.float32) \
            + b2[...]
        y2 = jnp.maximum(y2 * sc2 + sh2, 0.0)
        out[...] = jnp.max(y2.reshape(_CBLK, NSAMPLE, 128), axis=1)


@jax.jit
def _mlp_tc(grouped, E, W1p, b1, g1, be1, W2, b2, g2, be2):
    side = lambda shape: pl.BlockSpec(shape, lambda p, j: (0, 0))
    return pl.pallas_call(
        _mlp_body,
        grid=(3, _NBLK),
        in_specs=[
            pl.BlockSpec((_RBLK, _D), lambda p, j: (j, 0)),
            pl.BlockSpec((_CBLK, _D), lambda p, j: (j, 0)),
            side((_D, _C)), side((1, _C)), side((1, _C)), side((1, _C)),
            side((_C, 128)), side((1, 128)), side((1, 128)), side((1, 128)),
        ],
        out_specs=pl.BlockSpec((_CBLK, 128), lambda p, j: (j, 0)),
        out_shape=jax.ShapeDtypeStruct((_B * _S, 128), jnp.float32),
        scratch_shapes=[
            pltpu.VMEM((_D, _D), jnp.float32),
            pltpu.VMEM((_C, _C), jnp.float32),
            pltpu.VMEM((1, _C), jnp.float32),
        ],
    )(grouped, E, W1p, b1, g1, be1, W2, b2, g2, be2)


def kernel(xyz, features, num_points, W1, b1, g1, be1, W2, b2, g2, be2):
    B = xyz.shape[0]
    delta = (jnp.asarray(num_points) - NUM_POINTS).astype(jnp.int32)
    sample_idxs = jnp.tile(jnp.arange(NUM_POINTS, dtype=jnp.int32)[None, :],
                           (B, 1)) + delta
    new_xyz = xyz[:, :NUM_POINTS, :]

    xyzT = jnp.transpose(xyz, (0, 2, 1)).reshape(-1)
    feat_t = jnp.transpose(features, (0, 2, 1))
    T = jnp.concatenate(
        [xyz, feat_t, jnp.zeros((B, _N, _D - 3 - _C), jnp.float32)],
        axis=-1).reshape(B * _N, _D)

    grouped = _sc_group(xyzT, T)

    E = jnp.concatenate(
        [new_xyz, jnp.zeros((B, _S, _D - 4), jnp.float32),
         jnp.full((B, _S, 1), -1.0, jnp.float32)],
        axis=-1).reshape(B * _S, _D)
    W1p = jnp.concatenate(
        [W1, jnp.zeros((_D - 3 - _C, _C), jnp.float32)], axis=0)

    pooled = _mlp_tc(grouped, E, W1p, b1[None], g1[None], be1[None],
                     W2, b2[None], g2[None], be2[None])
    new_features = jnp.transpose(pooled.reshape(B, _S, 128), (0, 2, 1))
    return (new_xyz, new_features, sample_idxs)


# speedup vs baseline: 1.0072x; 1.0072x over previous
import functools

import jax
import jax.numpy as jnp
from jax import lax
from jax.experimental import pallas as pl
from jax.experimental.pallas import tpu as pltpu
from jax.experimental.pallas import tpu_sc as plsc

RADIUS = 0.2
NSAMPLE = 32
EPS = 1e-5
NUM_POINTS = 1024

_B, _N, _C = 4, 8192, 64
_S = NUM_POINTS
_D = 128
_NSC = 32
_GRP_PER_B = _NSC // _B
_S_PER_W = _S // _GRP_PER_B
_NCHUNK = _N // 16
_ROWS_PER_W = _S_PER_W * NSAMPLE
_GCHUNK = 128
_NGCH = _ROWS_PER_W // _GCHUNK
_P = _B * _S * NSAMPLE

_NCORES = 2
_NSUBCORES = 16
_UNROLL = 16
_MGRP = 16


def _bf16_round(x):
    u = lax.bitcast_convert_type(x, jnp.int32)
    lsb = jnp.bitwise_and(lax.shift_right_logical(u, 16), jnp.int32(1))
    r = u + lsb + jnp.int32(0x7FFF)
    return lax.bitcast_convert_type(jnp.bitwise_and(r, jnp.int32(-65536)),
                                    jnp.float32)


def _sc_body(xyzT_hbm, T_hbm, grouped_hbm,
             xv, yv, zv, xbv, ybv, zbv, pnv, idxv, rows0, rows1, sem0, sem1):
    wid = lax.axis_index("s") * _NCORES + lax.axis_index("c")
    b = wid // _GRP_PER_B
    g = wid % _GRP_PER_B
    pltpu.sync_copy(xyzT_hbm.at[pl.ds((b * 3 + 0) * _N, _N)], xv)
    pltpu.sync_copy(xyzT_hbm.at[pl.ds((b * 3 + 1) * _N, _N)], yv)
    pltpu.sync_copy(xyzT_hbm.at[pl.ds((b * 3 + 2) * _N, _N)], zv)
    s0 = g * _S_PER_W
    bN = b * _N
    lane = lax.iota(jnp.int32, 16)
    zero16 = jnp.zeros((16,), jnp.int32)
    r2 = jnp.full((16,), jnp.float32(RADIUS * RADIUS), jnp.float32)

    def prep(i, carry):
        x = xv[pl.ds(i * 16, 16)]
        y = yv[pl.ds(i * 16, 16)]
        z = zv[pl.ds(i * 16, 16)]
        pnv[pl.ds(i * 16, 16)] = x * x + y * y + z * z
        xbv[pl.ds(i * 16, 16)] = _bf16_round(x)
        ybv[pl.ds(i * 16, 16)] = _bf16_round(y)
        zbv[pl.ds(i * 16, 16)] = _bf16_round(z)
        return carry

    lax.fori_loop(0, _NCHUNK, prep, jnp.int32(0))

    def per_centroid(ci, carry):
        s = s0 + ci
        base = ci * NSAMPLE
        idxv[pl.ds(base, 16)] = zero16
        idxv[pl.ds(base + 16, 16)] = zero16
        calign = s0 + (ci // 16) * 16
        lsel = lane == (s - calign)
        zf = jnp.zeros((16,), jnp.float32)

        def extract(ref):
            v = ref[pl.ds(calign, 16)]
            return jnp.sum(jnp.where(lsel, v, zf))

        cx, cy, cz = extract(xv), extract(yv), extract(zv)
        cn = jnp.full((16,), cx * cx + cy * cy + cz * cz, jnp.float32)
        cbx = jnp.full((16,), 2.0 * extract(xbv), jnp.float32)
        cby = jnp.full((16,), 2.0 * extract(ybv), jnp.float32)
        cbz = jnp.full((16,), 2.0 * extract(zbv), jnp.float32)

        def cond(c):
            i, cnt = c
            return jnp.logical_and(i < _NCHUNK, cnt < NSAMPLE)

        def step(c):
            i, cnt = c
            for h in range(_UNROLL // _MGRP):
                ms = []
                for u in range(_MGRP):
                    ii = i + h * _MGRP + u
                    px = xbv[pl.ds(ii * 16, 16)]
                    py = ybv[pl.ds(ii * 16, 16)]
                    pz = zbv[pl.ds(ii * 16, 16)]
                    pn = pnv[pl.ds(ii * 16, 16)]
                    dot2 = (cbx * px + cby * py) + cbz * pz
                    d2 = (cn + pn) - dot2
                    ms.append(d2 < r2)
                for u in range(_MGRP):
                    ii = i + h * _MGRP + u
                    n = jnp.sum(ms[u].astype(jnp.int32))
                    plsc.store_compressed(idxv.at[pl.ds(base + cnt, 16)],
                                          lane + ii * 16, mask=ms[u])
                    cnt = cnt + n
            return (i + _UNROLL, cnt)

        _, cnt = lax.while_loop(cond, step, (jnp.int32(0), jnp.int32(0)))
        cntv = jnp.full((16,), cnt, jnp.int32)
        v0 = idxv[pl.ds(base, 16)]
        v1 = idxv[pl.ds(base + 16, 16)]
        zi = jnp.zeros((16,), jnp.int32)
        first = jnp.full((16,), jnp.sum(jnp.where(lane == 0, v0, zi)),
                         jnp.int32)
        bNv = jnp.full((16,), bN, jnp.int32)
        idxv[pl.ds(base, 16)] = jnp.where(lane < cntv, v0, first) + bNv
        idxv[pl.ds(base + 16, 16)] = jnp.where(lane + 16 < cntv, v1, first) \
            + bNv
        return carry

    lax.fori_loop(0, _S_PER_W, per_centroid, jnp.int32(0))

    row_base = wid * _ROWS_PER_W
    bufs = (rows0, rows1)
    sems = (sem0, sem1)

    def fire(j):
        return pltpu.async_copy(
            T_hbm.at[idxv.at[pl.ds(j * _GCHUNK, _GCHUNK)]],
            bufs[j % 2], sems[j % 2])

    dmas = [None, None]
    dmas[0] = fire(0)
    for j in range(_NGCH):
        dmas[j % 2].wait()
        if j + 1 < _NGCH:
            dmas[(j + 1) % 2] = fire(j + 1)
        pltpu.sync_copy(bufs[j % 2],
                        grouped_hbm.at[pl.ds(row_base + j * _GCHUNK,
                                             _GCHUNK)])


@jax.jit
def _sc_group(xyzT, T):
    mesh = plsc.VectorSubcoreMesh(core_axis_name="c", subcore_axis_name="s",
                                  num_cores=_NCORES, num_subcores=_NSUBCORES)
    f = functools.partial(
        pl.kernel,
        mesh=mesh,
        compiler_params=pltpu.CompilerParams(needs_layout_passes=False),
        out_type=jax.ShapeDtypeStruct((_B * _S * NSAMPLE, _D), jnp.float32),
        scratch_types=[
            pltpu.VMEM((_N,), jnp.float32),
            pltpu.VMEM((_N,), jnp.float32),
            pltpu.VMEM((_N,), jnp.float32),
            pltpu.VMEM((_N,), jnp.float32),
            pltpu.VMEM((_N,), jnp.float32),
            pltpu.VMEM((_N,), jnp.float32),
            pltpu.VMEM((_N,), jnp.float32),
            pltpu.VMEM((_ROWS_PER_W + 16 * _UNROLL + 32,), jnp.int32),
            pltpu.VMEM((_GCHUNK, _D), jnp.float32),
            pltpu.VMEM((_GCHUNK, _D), jnp.float32),
            pltpu.SemaphoreType.DMA,
            pltpu.SemaphoreType.DMA,
        ],
    )(_sc_body)
    return f(xyzT, T)


_RBLK = 4096
_CBLK = _RBLK // NSAMPLE
_NBLK = (_B * _S * NSAMPLE) // _RBLK


def _stats(W, bias, gamma, beta, G2, ssum):
    hp = jax.lax.Precision.HIGHEST
    sW = jnp.dot(ssum, W, precision=hp,
                 preferred_element_type=jnp.float32)
    GW = jnp.dot(G2, W, precision=hp,
                 preferred_element_type=jnp.float32)
    diag = jnp.sum(W * GW, axis=0, keepdims=True)
    Pf = jnp.float32(_P)
    mean = sW / Pf + bias
    ey2 = (diag + 2.0 * bias * sW) / Pf + bias * bias
    var = ey2 - mean * mean
    scale = gamma * jax.lax.rsqrt(var + jnp.float32(EPS))
    shift = beta - mean * scale
    return scale, shift


def _mlp_body(rows, E, W1, b1, g1, be1, W2, b2, g2, be2, out, Gs, Hs, s2s):
    p = pl.program_id(0)
    x = (rows[...].reshape(_CBLK, NSAMPLE, _D)
         - E[...][:, None, :]).reshape(_RBLK, _D)

    @pl.when(jnp.logical_and(p == 0, pl.program_id(1) == 0))
    def _():
        Gs[...] = jnp.zeros_like(Gs)

    @pl.when(p == 0)
    def _():
        Gs[...] += lax.dot_general(x, x, (((0,), (0,)), ((), ())),
                                   preferred_element_type=jnp.float32)

    @pl.when(jnp.logical_and(p == 1, pl.program_id(1) == 0))
    def _():
        Hs[...] = jnp.zeros_like(Hs)
        s2s[...] = jnp.zeros_like(s2s)

    def _a1(G):
        sc1, sh1 = _stats(W1[...], b1[...], g1[...], be1[...],
                          G, G[_D - 1:_D, :])
        y1 = jnp.dot(x, W1[...], preferred_element_type=jnp.float32) + b1[...]
        return jnp.maximum(y1 * sc1 + sh1, 0.0)

    @pl.when(p == 1)
    def _():
        a1 = _a1(Gs[...])
        Hs[...] += lax.dot_general(a1, a1, (((0,), (0,)), ((), ())),
                                   preferred_element_type=jnp.float32)
        s2s[...] += jnp.sum(a1, axis=0, keepdims=True)

    @pl.when(p == 2)
    def _():
        a1 = _a1(Gs[...])
        sc2, sh2 = _stats(W2[...], b2[...], g2[...], be2[...],
                          Hs[...], s2s[...])
        y2 = jnp.dot(a1, W2[...], preferred_element_type=jnp.float32) \
            + b2[...]
        y2 = jnp.maximum(y2 * sc2 + sh2, 0.0)
        out[...] = jnp.max(y2.reshape(_CBLK, NSAMPLE, 128), axis=1)


@jax.jit
def _mlp_tc(grouped, E, W1p, b1, g1, be1, W2, b2, g2, be2):
    side = lambda shape: pl.BlockSpec(shape, lambda p, j: (0, 0))
    return pl.pallas_call(
        _mlp_body,
        grid=(3, _NBLK),
        in_specs=[
            pl.BlockSpec((_RBLK, _D), lambda p, j: (j, 0)),
            pl.BlockSpec((_CBLK, _D), lambda p, j: (j, 0)),
            side((_D, _C)), side((1, _C)), side((1, _C)), side((1, _C)),
            side((_C, 128)), side((1, 128)), side((1, 128)), side((1, 128)),
        ],
        out_specs=pl.BlockSpec((_CBLK, 128), lambda p, j: (j, 0)),
        out_shape=jax.ShapeDtypeStruct((_B * _S, 128), jnp.float32),
        scratch_shapes=[
            pltpu.VMEM((_D, _D), jnp.float32),
            pltpu.VMEM((_C, _C), jnp.float32),
            pltpu.VMEM((1, _C), jnp.float32),
        ],
    )(grouped, E, W1p, b1, g1, be1, W2, b2, g2, be2)


def kernel(xyz, features, num_points, W1, b1, g1, be1, W2, b2, g2, be2):
    B = xyz.shape[0]
    delta = (jnp.asarray(num_points) - NUM_POINTS).astype(jnp.int32)
    sample_idxs = jnp.tile(jnp.arange(NUM_POINTS, dtype=jnp.int32)[None, :],
                           (B, 1)) + delta
    new_xyz = xyz[:, :NUM_POINTS, :]

    xyzT = jnp.transpose(xyz, (0, 2, 1)).reshape(-1)
    feat_t = jnp.transpose(features, (0, 2, 1))
    T = jnp.concatenate(
        [xyz, feat_t, jnp.zeros((B, _N, _D - 3 - _C), jnp.float32)],
        axis=-1).reshape(B * _N, _D)

    grouped = _sc_group(xyzT, T)

    E = jnp.concatenate(
        [new_xyz, jnp.zeros((B, _S, _D - 4), jnp.float32),
         jnp.full((B, _S, 1), -1.0, jnp.float32)],
        axis=-1).reshape(B * _S, _D)
    W1p = jnp.concatenate(
        [W1, jnp.zeros((_D - 3 - _C, _C), jnp.float32)], axis=0)

    pooled = _mlp_tc(grouped, E, W1p, b1[None], g1[None], be1[None],
                     W2, b2[None], g2[None], be2[None])
    new_features = jnp.transpose(pooled.reshape(B, _S, 128), (0, 2, 1))
    return (new_xyz, new_features, sample_idxs)


# speedup vs baseline: 1.1212x; 1.1131x over previous
import functools

import jax
import jax.numpy as jnp
from jax import lax
from jax.experimental import pallas as pl
from jax.experimental.pallas import tpu as pltpu
from jax.experimental.pallas import tpu_sc as plsc

RADIUS = 0.2
NSAMPLE = 32
EPS = 1e-5
NUM_POINTS = 1024

_B, _N, _C = 4, 8192, 64
_S = NUM_POINTS
_D = 128
_NSC = 32
_GRP_PER_B = _NSC // _B
_S_PER_W = _S // _GRP_PER_B
_NCHUNK = _N // 16
_ROWS_PER_W = _S_PER_W * NSAMPLE
_GCHUNK = 128
_NGCH = _ROWS_PER_W // _GCHUNK
_P = _B * _S * NSAMPLE

_NCORES = 2
_NSUBCORES = 16
_UNROLL = 16
_MGRP = 16


def _bf16_round(x):
    u = lax.bitcast_convert_type(x, jnp.int32)
    lsb = jnp.bitwise_and(lax.shift_right_logical(u, 16), jnp.int32(1))
    r = u + lsb + jnp.int32(0x7FFF)
    return lax.bitcast_convert_type(jnp.bitwise_and(r, jnp.int32(-65536)),
                                    jnp.float32)


def _sc_body(xyzT_hbm, T_hbm, grouped_hbm,
             xv, yv, zv, xbv, ybv, zbv, pnv, idxv, rows0, rows1, sem0, sem1):
    wid = lax.axis_index("s") * _NCORES + lax.axis_index("c")
    b = wid // _GRP_PER_B
    g = wid % _GRP_PER_B
    pltpu.sync_copy(xyzT_hbm.at[pl.ds((b * 3 + 0) * _N, _N)], xv)
    pltpu.sync_copy(xyzT_hbm.at[pl.ds((b * 3 + 1) * _N, _N)], yv)
    pltpu.sync_copy(xyzT_hbm.at[pl.ds((b * 3 + 2) * _N, _N)], zv)
    s0 = g * _S_PER_W
    bN = b * _N
    lane = lax.iota(jnp.int32, 16)
    zero16 = jnp.zeros((16,), jnp.int32)
    r2 = jnp.full((16,), jnp.float32(RADIUS * RADIUS), jnp.float32)

    def prep(i, carry):
        x = xv[pl.ds(i * 16, 16)]
        y = yv[pl.ds(i * 16, 16)]
        z = zv[pl.ds(i * 16, 16)]
        pnv[pl.ds(i * 16, 16)] = x * x + y * y + z * z
        xbv[pl.ds(i * 16, 16)] = _bf16_round(x)
        ybv[pl.ds(i * 16, 16)] = _bf16_round(y)
        zbv[pl.ds(i * 16, 16)] = _bf16_round(z)
        return carry

    lax.fori_loop(0, _NCHUNK, prep, jnp.int32(0))

    def per_centroid(ci, carry):
        s = s0 + ci
        base = ci * NSAMPLE
        idxv[pl.ds(base, 16)] = zero16
        idxv[pl.ds(base + 16, 16)] = zero16
        calign = s0 + (ci // 16) * 16
        lsel = lane == (s - calign)
        zf = jnp.zeros((16,), jnp.float32)

        def extract(ref):
            v = ref[pl.ds(calign, 16)]
            return jnp.sum(jnp.where(lsel, v, zf))

        cx, cy, cz = extract(xv), extract(yv), extract(zv)
        cn = jnp.full((16,), cx * cx + cy * cy + cz * cz, jnp.float32)
        cbx = jnp.full((16,), 2.0 * extract(xbv), jnp.float32)
        cby = jnp.full((16,), 2.0 * extract(ybv), jnp.float32)
        cbz = jnp.full((16,), 2.0 * extract(zbv), jnp.float32)

        def cond(c):
            i, cnt = c
            return jnp.logical_and(i < _NCHUNK, cnt < NSAMPLE)

        def step(c):
            i, cnt = c
            for h in range(_UNROLL // _MGRP):
                ms = []
                for u in range(_MGRP):
                    ii = i + h * _MGRP + u
                    px = xbv[pl.ds(ii * 16, 16)]
                    py = ybv[pl.ds(ii * 16, 16)]
                    pz = zbv[pl.ds(ii * 16, 16)]
                    pn = pnv[pl.ds(ii * 16, 16)]
                    dot2 = (cbx * px + cby * py) + cbz * pz
                    d2 = (cn + pn) - dot2
                    ms.append(d2 < r2)
                for u in range(_MGRP):
                    ii = i + h * _MGRP + u
                    n = jnp.sum(ms[u].astype(jnp.int32))
                    plsc.store_compressed(idxv.at[pl.ds(base + cnt, 16)],
                                          lane + ii * 16, mask=ms[u])
                    cnt = cnt + n
            return (i + _UNROLL, cnt)

        _, cnt = lax.while_loop(cond, step, (jnp.int32(0), jnp.int32(0)))
        cntv = jnp.full((16,), cnt, jnp.int32)
        v0 = idxv[pl.ds(base, 16)]
        v1 = idxv[pl.ds(base + 16, 16)]
        zi = jnp.zeros((16,), jnp.int32)
        first = jnp.full((16,), jnp.sum(jnp.where(lane == 0, v0, zi)),
                         jnp.int32)
        bNv = jnp.full((16,), bN, jnp.int32)
        idxv[pl.ds(base, 16)] = jnp.where(lane < cntv, v0, first) + bNv
        idxv[pl.ds(base + 16, 16)] = jnp.where(lane + 16 < cntv, v1, first) \
            + bNv
        return carry

    lax.fori_loop(0, _S_PER_W, per_centroid, jnp.int32(0))

    row_base = wid * _ROWS_PER_W
    bufs = (rows0, rows1)
    sems = (sem0, sem1)

    def fire(j):
        return pltpu.async_copy(
            T_hbm.at[idxv.at[pl.ds(j * _GCHUNK, _GCHUNK)]],
            bufs[j % 2], sems[j % 2])

    dmas = [None, None]
    dmas[0] = fire(0)
    for j in range(_NGCH):
        dmas[j % 2].wait()
        if j + 1 < _NGCH:
            dmas[(j + 1) % 2] = fire(j + 1)
        pltpu.sync_copy(bufs[j % 2],
                        grouped_hbm.at[pl.ds(row_base + j * _GCHUNK,
                                             _GCHUNK)])


@jax.jit
def _sc_group(xyzT, T):
    mesh = plsc.VectorSubcoreMesh(core_axis_name="c", subcore_axis_name="s",
                                  num_cores=_NCORES, num_subcores=_NSUBCORES)
    f = functools.partial(
        pl.kernel,
        mesh=mesh,
        compiler_params=pltpu.CompilerParams(needs_layout_passes=False),
        out_type=jax.ShapeDtypeStruct((_B * _S * NSAMPLE, _D), jnp.float32),
        scratch_types=[
            pltpu.VMEM((_N,), jnp.float32),
            pltpu.VMEM((_N,), jnp.float32),
            pltpu.VMEM((_N,), jnp.float32),
            pltpu.VMEM((_N,), jnp.float32),
            pltpu.VMEM((_N,), jnp.float32),
            pltpu.VMEM((_N,), jnp.float32),
            pltpu.VMEM((_N,), jnp.float32),
            pltpu.VMEM((_ROWS_PER_W + 16 * _UNROLL + 32,), jnp.int32),
            pltpu.VMEM((_GCHUNK, _D), jnp.float32),
            pltpu.VMEM((_GCHUNK, _D), jnp.float32),
            pltpu.SemaphoreType.DMA,
            pltpu.SemaphoreType.DMA,
        ],
    )(_sc_body)
    return f(xyzT, T)


_RBLK = 8192
_CBLK = _RBLK // NSAMPLE
_NBLK = (_B * _S * NSAMPLE) // _RBLK


def _stats(W, bias, gamma, beta, G2, ssum):
    hp = jax.lax.Precision.HIGHEST
    sW = jnp.dot(ssum, W, precision=hp,
                 preferred_element_type=jnp.float32)
    GW = jnp.dot(G2, W, precision=hp,
                 preferred_element_type=jnp.float32)
    diag = jnp.sum(W * GW, axis=0, keepdims=True)
    Pf = jnp.float32(_P)
    mean = sW / Pf + bias
    ey2 = (diag + 2.0 * bias * sW) / Pf + bias * bias
    var = ey2 - mean * mean
    scale = gamma * jax.lax.rsqrt(var + jnp.float32(EPS))
    shift = beta - mean * scale
    return scale, shift


def _mlp_body(rows, E, W1, b1, g1, be1, W2, b2, g2, be2, out, Gs, Hs, s2s):
    p = pl.program_id(0)
    x = (rows[...].reshape(_CBLK, NSAMPLE, _D)
         - E[...][:, None, :]).reshape(_RBLK, _D)

    @pl.when(jnp.logical_and(p == 0, pl.program_id(1) == 0))
    def _():
        Gs[...] = jnp.zeros_like(Gs)

    @pl.when(p == 0)
    def _():
        Gs[...] += lax.dot_general(x, x, (((0,), (0,)), ((), ())),
                                   preferred_element_type=jnp.float32)

    @pl.when(jnp.logical_and(p == 1, pl.program_id(1) == 0))
    def _():
        Hs[...] = jnp.zeros_like(Hs)
        s2s[...] = jnp.zeros_like(s2s)

    def _a1(G):
        sc1, sh1 = _stats(W1[...], b1[...], g1[...], be1[...],
                          G, G[_D - 1:_D, :])
        y1 = jnp.dot(x, W1[...], preferred_element_type=jnp.float32) + b1[...]
        return jnp.maximum(y1 * sc1 + sh1, 0.0)

    @pl.when(p == 1)
    def _():
        a1 = _a1(Gs[...])
        Hs[...] += lax.dot_general(a1, a1, (((0,), (0,)), ((), ())),
                                   preferred_element_type=jnp.float32)
        s2s[...] += jnp.sum(a1, axis=0, keepdims=True)

    @pl.when(p == 2)
    def _():
        a1 = _a1(Gs[...])
        sc2, sh2 = _stats(W2[...], b2[...], g2[...], be2[...],
                          Hs[...], s2s[...])
        y2 = jnp.dot(a1, W2[...], preferred_element_type=jnp.float32) \
            + b2[...]
        y2 = jnp.maximum(y2 * sc2 + sh2, 0.0)
        out[...] = jnp.max(y2.reshape(_CBLK, NSAMPLE, 128), axis=1)


@jax.jit
def _mlp_tc(grouped, E, W1p, b1, g1, be1, W2, b2, g2, be2):
    side = lambda shape: pl.BlockSpec(shape, lambda p, j: (0, 0))
    return pl.pallas_call(
        _mlp_body,
        grid=(3, _NBLK),
        in_specs=[
            pl.BlockSpec((_RBLK, _D), lambda p, j: (j, 0)),
            pl.BlockSpec((_CBLK, _D), lambda p, j: (j, 0)),
            side((_D, _C)), side((1, _C)), side((1, _C)), side((1, _C)),
            side((_C, 128)), side((1, 128)), side((1, 128)), side((1, 128)),
        ],
        out_specs=pl.BlockSpec((_CBLK, 128), lambda p, j: (j, 0)),
        out_shape=jax.ShapeDtypeStruct((_B * _S, 128), jnp.float32),
        scratch_shapes=[
            pltpu.VMEM((_D, _D), jnp.float32),
            pltpu.VMEM((_C, _C), jnp.float32),
            pltpu.VMEM((1, _C), jnp.float32),
        ],
    )(grouped, E, W1p, b1, g1, be1, W2, b2, g2, be2)


def kernel(xyz, features, num_points, W1, b1, g1, be1, W2, b2, g2, be2):
    B = xyz.shape[0]
    delta = (jnp.asarray(num_points) - NUM_POINTS).astype(jnp.int32)
    sample_idxs = jnp.tile(jnp.arange(NUM_POINTS, dtype=jnp.int32)[None, :],
                           (B, 1)) + delta
    new_xyz = xyz[:, :NUM_POINTS, :]

    xyzT = jnp.transpose(xyz, (0, 2, 1)).reshape(-1)
    feat_t = jnp.transpose(features, (0, 2, 1))
    T = jnp.concatenate(
        [xyz, feat_t, jnp.zeros((B, _N, _D - 3 - _C), jnp.float32)],
        axis=-1).reshape(B * _N, _D)

    grouped = _sc_group(xyzT, T)

    E = jnp.concatenate(
        [new_xyz, jnp.zeros((B, _S, _D - 4), jnp.float32),
         jnp.full((B, _S, 1), -1.0, jnp.float32)],
        axis=-1).reshape(B * _S, _D)
    W1p = jnp.concatenate(
        [W1, jnp.zeros((_D - 3 - _C, _C), jnp.float32)], axis=0)

    pooled = _mlp_tc(grouped, E, W1p, b1[None], g1[None], be1[None],
                     W2, b2[None], g2[None], be2[None])
    new_features = jnp.transpose(pooled.reshape(B, _S, 128), (0, 2, 1))
    return (new_xyz, new_features, sample_idxs)


# speedup vs baseline: 1.1970x; 1.0676x over previous
import functools

import jax
import jax.numpy as jnp
from jax import lax
from jax.experimental import pallas as pl
from jax.experimental.pallas import tpu as pltpu
from jax.experimental.pallas import tpu_sc as plsc

RADIUS = 0.2
NSAMPLE = 32
EPS = 1e-5
NUM_POINTS = 1024

_B, _N, _C = 4, 8192, 64
_S = NUM_POINTS
_D = 128
_NSC = 32
_GRP_PER_B = _NSC // _B
_S_PER_W = _S // _GRP_PER_B
_NCHUNK = _N // 16
_ROWS_PER_W = _S_PER_W * NSAMPLE
_GCHUNK = 128
_GC = 8
_P = _B * _S * NSAMPLE

_NCORES = 2
_NSUBCORES = 16
_UNROLL = 16
_MGRP = 16


def _bf16_round(x):
    u = lax.bitcast_convert_type(x, jnp.int32)
    lsb = jnp.bitwise_and(lax.shift_right_logical(u, 16), jnp.int32(1))
    r = u + lsb + jnp.int32(0x7FFF)
    return lax.bitcast_convert_type(jnp.bitwise_and(r, jnp.int32(-65536)),
                                    jnp.float32)


def _sc_body(xyzT_hbm, T_hbm, grouped_hbm,
             xv, yv, zv, xbv, ybv, zbv, pnv, idxv, rows0, rows1,
             sem0, sem1, wsem0, wsem1):
    wid = lax.axis_index("s") * _NCORES + lax.axis_index("c")
    b = wid // _GRP_PER_B
    g = wid % _GRP_PER_B
    pltpu.sync_copy(xyzT_hbm.at[pl.ds((b * 3 + 0) * _N, _N)], xv)
    pltpu.sync_copy(xyzT_hbm.at[pl.ds((b * 3 + 1) * _N, _N)], yv)
    pltpu.sync_copy(xyzT_hbm.at[pl.ds((b * 3 + 2) * _N, _N)], zv)
    s0 = g * _S_PER_W
    bN = b * _N
    lane = lax.iota(jnp.int32, 16)
    zero16 = jnp.zeros((16,), jnp.int32)
    r2 = jnp.full((16,), jnp.float32(RADIUS * RADIUS), jnp.float32)

    def prep(i, carry):
        x = xv[pl.ds(i * 16, 16)]
        y = yv[pl.ds(i * 16, 16)]
        z = zv[pl.ds(i * 16, 16)]
        pnv[pl.ds(i * 16, 16)] = x * x + y * y + z * z
        xbv[pl.ds(i * 16, 16)] = _bf16_round(x)
        ybv[pl.ds(i * 16, 16)] = _bf16_round(y)
        zbv[pl.ds(i * 16, 16)] = _bf16_round(z)
        return carry

    lax.fori_loop(0, _NCHUNK, prep, jnp.int32(0))

    def per_centroid(ci, carry):
        s = s0 + ci
        base = ci * NSAMPLE
        idxv[pl.ds(base, 16)] = zero16
        idxv[pl.ds(base + 16, 16)] = zero16
        calign = s0 + (ci // 16) * 16
        lsel = lane == (s - calign)
        zf = jnp.zeros((16,), jnp.float32)

        def extract(ref):
            v = ref[pl.ds(calign, 16)]
            return jnp.sum(jnp.where(lsel, v, zf))

        cx, cy, cz = extract(xv), extract(yv), extract(zv)
        cn = jnp.full((16,), cx * cx + cy * cy + cz * cz, jnp.float32)
        cbx = jnp.full((16,), 2.0 * extract(xbv), jnp.float32)
        cby = jnp.full((16,), 2.0 * extract(ybv), jnp.float32)
        cbz = jnp.full((16,), 2.0 * extract(zbv), jnp.float32)

        def cond(c):
            i, cnt = c
            return jnp.logical_and(i < _NCHUNK, cnt < NSAMPLE)

        def step(c):
            i, cnt = c
            for h in range(_UNROLL // _MGRP):
                ms = []
                for u in range(_MGRP):
                    ii = i + h * _MGRP + u
                    px = xbv[pl.ds(ii * 16, 16)]
                    py = ybv[pl.ds(ii * 16, 16)]
                    pz = zbv[pl.ds(ii * 16, 16)]
                    pn = pnv[pl.ds(ii * 16, 16)]
                    dot2 = (cbx * px + cby * py) + cbz * pz
                    d2 = (cn + pn) - dot2
                    ms.append(d2 < r2)
                for u in range(_MGRP):
                    ii = i + h * _MGRP + u
                    n = jnp.sum(ms[u].astype(jnp.int32))
                    plsc.store_compressed(idxv.at[pl.ds(base + cnt, 16)],
                                          lane + ii * 16, mask=ms[u])
                    cnt = cnt + n
            return (i + _UNROLL, cnt)

        _, cnt = lax.while_loop(cond, step, (jnp.int32(0), jnp.int32(0)))
        cntv = jnp.full((16,), cnt, jnp.int32)
        v0 = idxv[pl.ds(base, 16)]
        v1 = idxv[pl.ds(base + 16, 16)]
        zi = jnp.zeros((16,), jnp.int32)
        first = jnp.full((16,), jnp.sum(jnp.where(lane == 0, v0, zi)),
                         jnp.int32)
        bNv = jnp.full((16,), bN, jnp.int32)
        idxv[pl.ds(base, 16)] = jnp.where(lane < cntv, v0, first) + bNv
        idxv[pl.ds(base + 16, 16)] = jnp.where(lane + 16 < cntv, v1, first) \
            + bNv
        return carry

    row_base = wid * _ROWS_PER_W
    bufs = (rows0, rows1)
    gsems = (sem0, sem1)
    wsems = (wsem0, wsem1)
    grows = _GC * NSAMPLE
    gds = [None, None]
    wrs = [None, None]

    def fire(g2):
        p = g2 % 2
        i0 = g2 * grows
        return [
            pltpu.async_copy(
                T_hbm.at[idxv.at[pl.ds(i0 + k * _GCHUNK, _GCHUNK)]],
                bufs[p].at[pl.ds(k * _GCHUNK, _GCHUNK)], gsems[p])
            for k in range(grows // _GCHUNK)
        ]

    def writeout(g2):
        p = g2 % 2
        return pltpu.async_copy(
            bufs[p],
            grouped_hbm.at[pl.ds(row_base + g2 * grows, grows)], wsems[p])

    for g2 in range(_S_PER_W // _GC):
        lax.fori_loop(g2 * _GC, (g2 + 1) * _GC, per_centroid, jnp.int32(0))
        if g2 >= 2:
            wrs[g2 % 2].wait()
        gds[g2 % 2] = fire(g2)
        if g2 >= 1:
            for d in gds[(g2 - 1) % 2]:
                d.wait()
            wrs[(g2 - 1) % 2] = writeout(g2 - 1)
    last = _S_PER_W // _GC - 1
    for d in gds[last % 2]:
        d.wait()
    wrs[last % 2] = writeout(last)
    wrs[(last - 1) % 2].wait()
    wrs[last % 2].wait()


@jax.jit
def _sc_group(xyzT, T):
    mesh = plsc.VectorSubcoreMesh(core_axis_name="c", subcore_axis_name="s",
                                  num_cores=_NCORES, num_subcores=_NSUBCORES)
    f = functools.partial(
        pl.kernel,
        mesh=mesh,
        compiler_params=pltpu.CompilerParams(needs_layout_passes=False),
        out_type=jax.ShapeDtypeStruct((_B * _S * NSAMPLE, _D), jnp.float32),
        scratch_types=[
            pltpu.VMEM((_N,), jnp.float32),
            pltpu.VMEM((_N,), jnp.float32),
            pltpu.VMEM((_N,), jnp.float32),
            pltpu.VMEM((_N,), jnp.float32),
            pltpu.VMEM((_N,), jnp.float32),
            pltpu.VMEM((_N,), jnp.float32),
            pltpu.VMEM((_N,), jnp.float32),
            pltpu.VMEM((_ROWS_PER_W + 16 * _UNROLL + 32,), jnp.int32),
            pltpu.VMEM((_GC * NSAMPLE, _D), jnp.float32),
            pltpu.VMEM((_GC * NSAMPLE, _D), jnp.float32),
            pltpu.SemaphoreType.DMA,
            pltpu.SemaphoreType.DMA,
            pltpu.SemaphoreType.DMA,
            pltpu.SemaphoreType.DMA,
        ],
    )(_sc_body)
    return f(xyzT, T)


_RBLK = 4096
_CBLK = _RBLK // NSAMPLE
_NBLK = (_B * _S * NSAMPLE) // _RBLK


def _stats(W, bias, gamma, beta, G2, ssum):
    hp = jax.lax.Precision.HIGHEST
    sW = jnp.dot(ssum, W, precision=hp,
                 preferred_element_type=jnp.float32)
    GW = jnp.dot(G2, W, precision=hp,
                 preferred_element_type=jnp.float32)
    diag = jnp.sum(W * GW, axis=0, keepdims=True)
    Pf = jnp.float32(_P)
    mean = sW / Pf + bias
    ey2 = (diag + 2.0 * bias * sW) / Pf + bias * bias
    var = ey2 - mean * mean
    scale = gamma * jax.lax.rsqrt(var + jnp.float32(EPS))
    shift = beta - mean * scale
    return scale, shift


def _mlp_body(rows, E, W1, b1, g1, be1, W2, b2, g2, be2, out, Gs, Hs, s2s):
    p = pl.program_id(0)
    x = (rows[...].reshape(_CBLK, NSAMPLE, _D)
         - E[...][:, None, :]).reshape(_RBLK, _D)

    @pl.when(jnp.logical_and(p == 0, pl.program_id(1) == 0))
    def _():
        Gs[...] = jnp.zeros_like(Gs)

    @pl.when(p == 0)
    def _():
        Gs[...] += lax.dot_general(x, x, (((0,), (0,)), ((), ())),
                                   preferred_element_type=jnp.float32)

    @pl.when(jnp.logical_and(p == 1, pl.program_id(1) == 0))
    def _():
        Hs[...] = jnp.zeros_like(Hs)
        s2s[...] = jnp.zeros_like(s2s)

    def _a1(G):
        sc1, sh1 = _stats(W1[...], b1[...], g1[...], be1[...],
                          G, G[_D - 1:_D, :])
        y1 = jnp.dot(x, W1[...], preferred_element_type=jnp.float32) + b1[...]
        return jnp.maximum(y1 * sc1 + sh1, 0.0)

    @pl.when(p == 1)
    def _():
        a1 = _a1(Gs[...])
        Hs[...] += lax.dot_general(a1, a1, (((0,), (0,)), ((), ())),
                                   preferred_element_type=jnp.float32)
        s2s[...] += jnp.sum(a1, axis=0, keepdims=True)

    @pl.when(p == 2)
    def _():
        a1 = _a1(Gs[...])
        sc2, sh2 = _stats(W2[...], b2[...], g2[...], be2[...],
                          Hs[...], s2s[...])
        y2 = jnp.dot(a1, W2[...], preferred_element_type=jnp.float32) \
            + b2[...]
        y2 = jnp.maximum(y2 * sc2 + sh2, 0.0)
        out[...] = jnp.max(y2.reshape(_CBLK, NSAMPLE, 128), axis=1)


@jax.jit
def _mlp_tc(grouped, E, W1p, b1, g1, be1, W2, b2, g2, be2):
    side = lambda shape: pl.BlockSpec(shape, lambda p, j: (0, 0))
    return pl.pallas_call(
        _mlp_body,
        grid=(3, _NBLK),
        in_specs=[
            pl.BlockSpec((_RBLK, _D), lambda p, j: (j, 0)),
            pl.BlockSpec((_CBLK, _D), lambda p, j: (j, 0)),
            side((_D, _C)), side((1, _C)), side((1, _C)), side((1, _C)),
            side((_C, 128)), side((1, 128)), side((1, 128)), side((1, 128)),
        ],
        out_specs=pl.BlockSpec((_CBLK, 128), lambda p, j: (j, 0)),
        out_shape=jax.ShapeDtypeStruct((_B * _S, 128), jnp.float32),
        scratch_shapes=[
            pltpu.VMEM((_D, _D), jnp.float32),
            pltpu.VMEM((_C, _C), jnp.float32),
            pltpu.VMEM((1, _C), jnp.float32),
        ],
    )(grouped, E, W1p, b1, g1, be1, W2, b2, g2, be2)


def kernel(xyz, features, num_points, W1, b1, g1, be1, W2, b2, g2, be2):
    B = xyz.shape[0]
    delta = (jnp.asarray(num_points) - NUM_POINTS).astype(jnp.int32)
    sample_idxs = jnp.tile(jnp.arange(NUM_POINTS, dtype=jnp.int32)[None, :],
                           (B, 1)) + delta
    new_xyz = xyz[:, :NUM_POINTS, :]

    xyzT = jnp.transpose(xyz, (0, 2, 1)).reshape(-1)
    feat_t = jnp.transpose(features, (0, 2, 1))
    T = jnp.concatenate(
        [xyz, feat_t, jnp.zeros((B, _N, _D - 3 - _C), jnp.float32)],
        axis=-1).reshape(B * _N, _D)

    grouped = _sc_group(xyzT, T)

    E = jnp.concatenate(
        [new_xyz, jnp.zeros((B, _S, _D - 4), jnp.float32),
         jnp.full((B, _S, 1), -1.0, jnp.float32)],
        axis=-1).reshape(B * _S, _D)
    W1p = jnp.concatenate(
        [W1, jnp.zeros((_D - 3 - _C, _C), jnp.float32)], axis=0)

    pooled = _mlp_tc(grouped, E, W1p, b1[None], g1[None], be1[None],
                     W2, b2[None], g2[None], be2[None])
    new_features = jnp.transpose(pooled.reshape(B, _S, 128), (0, 2, 1))
    return (new_xyz, new_features, sample_idxs)
